# Initial kernel scaffold; baseline (speedup 1.0000x reference)
import functools

import jax
import jax.numpy as jnp
from jax import lax
from jax.experimental import pallas as pl
from jax.experimental.pallas import tpu as pltpu
from jax.experimental.pallas import tpu_sc as plsc

D = 64
LANES = 16
NC = 2
NS = 16
NW = NC * NS
CHUNK = 128


def _tile_body(ph, pr, pt, nh, nr, nt, ent_w, rel_w, norm_w,
               pos_out, neg_out,
               ph_v, pr_v, pt_v, nh_v, nr_v, nt_v,
               hbuf0, hbuf1, tbuf0, tbuf1, rbuf0, rbuf1, nbuf0, nbuf1,
               opos_v, oneg_v, sem0, sem1):
    nch = ph_v.shape[0]
    groups = CHUNK // LANES

    cid = lax.axis_index("c")
    sid = lax.axis_index("s")
    wid = sid * NC + cid

    pltpu.sync_copy(ph.at[wid], ph_v)
    pltpu.sync_copy(pr.at[wid], pr_v)
    pltpu.sync_copy(pt.at[wid], pt_v)
    pltpu.sync_copy(nh.at[wid], nh_v)
    pltpu.sync_copy(nr.at[wid], nr_v)
    pltpu.sync_copy(nt.at[wid], nt_v)

    hbufs = (hbuf0, hbuf1)
    tbufs = (tbuf0, tbuf1)
    rbufs = (rbuf0, rbuf1)
    nbufs = (nbuf0, nbuf1)
    sems = (sem0, sem1)
    triples = [(ph_v, pr_v, pt_v)] * nch + [(nh_v, nr_v, nt_v)] * nch
    outs = [opos_v] * nch + [oneg_v] * nch
    nitems = 2 * nch

    def start(i):
        b = i % 2
        hv, rv, tv = triples[i]
        ci = i % nch
        return [
            pltpu.async_copy(ent_w.at[hv.at[ci]], hbufs[b], sems[b]),
            pltpu.async_copy(ent_w.at[tv.at[ci]], tbufs[b], sems[b]),
            pltpu.async_copy(rel_w.at[rv.at[ci]], rbufs[b], sems[b]),
            pltpu.async_copy(norm_w.at[rv.at[ci]], nbufs[b], sems[b]),
        ]

    lanes = lax.iota(jnp.int32, LANES)
    zeros = jnp.zeros((LANES,), jnp.float32)
    pend = start(0)
    for i in range(nitems):
        nxt = start(i + 1) if i + 1 < nitems else []
        for c in pend:
            c.wait()
        pend = nxt
        b = i % 2
        hb, tb, rb, nb = hbufs[b], tbufs[b], rbufs[b], nbufs[b]
        obuf = outs[i]
        obase = (i % nch) * CHUNK

        def group(g, _, hb=hb, tb=tb, rb=rb, nb=nb, obuf=obuf, obase=obase):
            rows = lanes + g * LANES

            def dstep(d, carry):
                a, c2, s, n2 = carry
                cols = jnp.zeros((LANES,), jnp.int32) + d
                hvv = plsc.load_gather(hb, [rows, cols])
                tvv = plsc.load_gather(tb, [rows, cols])
                rvv = plsc.load_gather(rb, [rows, cols])
                nvv = plsc.load_gather(nb, [rows, cols])
                u = hvv - tvv
                w = u + rvv
                return (a + w * w, c2 + w * nvv, s + u * nvv, n2 + nvv * nvv)

            a, c2, s, n2 = lax.fori_loop(0, D, dstep, (zeros,) * 4)
            res = a - 2.0 * s * c2 + s * s * n2
            obuf[pl.ds(obase + g * LANES, LANES)] = res
            return 0

        lax.fori_loop(0, groups, group, 0)

    pltpu.sync_copy(opos_v, pos_out.at[wid])
    pltpu.sync_copy(oneg_v, neg_out.at[wid])


def kernel(ph, pr, pt, nh, nr, nt, ent_w, rel_w, norm_w):
    B = ph.shape[0]
    rows_per_w = B // NW
    nch = rows_per_w // CHUNK

    mesh = plsc.VectorSubcoreMesh(core_axis_name="c", subcore_axis_name="s")
    run = functools.partial(
        pl.kernel,
        mesh=mesh,
        out_type=[
            jax.ShapeDtypeStruct((NW, rows_per_w), jnp.float32),
            jax.ShapeDtypeStruct((NW, rows_per_w), jnp.float32),
        ],
        scratch_types=[
            pltpu.VMEM((nch, CHUNK), jnp.int32),
            pltpu.VMEM((nch, CHUNK), jnp.int32),
            pltpu.VMEM((nch, CHUNK), jnp.int32),
            pltpu.VMEM((nch, CHUNK), jnp.int32),
            pltpu.VMEM((nch, CHUNK), jnp.int32),
            pltpu.VMEM((nch, CHUNK), jnp.int32),
            pltpu.VMEM((CHUNK, D), jnp.float32),
            pltpu.VMEM((CHUNK, D), jnp.float32),
            pltpu.VMEM((CHUNK, D), jnp.float32),
            pltpu.VMEM((CHUNK, D), jnp.float32),
            pltpu.VMEM((CHUNK, D), jnp.float32),
            pltpu.VMEM((CHUNK, D), jnp.float32),
            pltpu.VMEM((CHUNK, D), jnp.float32),
            pltpu.VMEM((CHUNK, D), jnp.float32),
            pltpu.VMEM((rows_per_w,), jnp.float32),
            pltpu.VMEM((rows_per_w,), jnp.float32),
            pltpu.SemaphoreType.DMA,
            pltpu.SemaphoreType.DMA,
        ],
    )(_tile_body)

    pos, neg = run(
        ph.reshape(NW, nch, CHUNK), pr.reshape(NW, nch, CHUNK),
        pt.reshape(NW, nch, CHUNK), nh.reshape(NW, nch, CHUNK),
        nr.reshape(NW, nch, CHUNK), nt.reshape(NW, nch, CHUNK),
        ent_w, rel_w, norm_w,
    )
    return pos.reshape(B), neg.reshape(B)


# speedup vs baseline: 1.0288x; 1.0288x over previous
import functools

import jax
import jax.numpy as jnp
from jax import lax
from jax.experimental import pallas as pl
from jax.experimental.pallas import tpu as pltpu
from jax.experimental.pallas import tpu_sc as plsc

D = 64
LANES = 16
NC = 2
NS = 16
NW = NC * NS
CHUNK = 64
UNROLL = 8


def _tile_body(ph, pr, pt, nh, nr, nt, ent_w, rel_w, norm_w,
               pos_out, neg_out,
               ph_v, pr_v, pt_v, nh_v, nr_v, nt_v,
               shidx,
               hs0, hs1, rs0, rs1, ts0, ts1,
               hbuf0, hbuf1, tbuf0, tbuf1, rbuf0, rbuf1, nbuf0, nbuf1,
               opos_v, oneg_v, sem0, sem1):
    groups = CHUNK // LANES

    cid = lax.axis_index("c")
    sid = lax.axis_index("s")
    wid = sid * NC + cid

    idx_copies = [
        pltpu.async_copy(ph.at[wid], ph_v, sem0),
        pltpu.async_copy(pr.at[wid], pr_v, sem0),
        pltpu.async_copy(pt.at[wid], pt_v, sem0),
        pltpu.async_copy(nh.at[wid], nh_v, sem0),
        pltpu.async_copy(nr.at[wid], nr_v, sem0),
        pltpu.async_copy(nt.at[wid], nt_v, sem0),
    ]
    for c in idx_copies:
        c.wait()

    pltpu.sync_copy(ph_v, shidx.at[sid, 0])
    pltpu.sync_copy(pr_v, shidx.at[sid, 1])
    pltpu.sync_copy(pt_v, shidx.at[sid, 2])
    pltpu.sync_copy(nh_v, shidx.at[sid, 3])
    pltpu.sync_copy(nr_v, shidx.at[sid, 4])
    pltpu.sync_copy(nt_v, shidx.at[sid, 5])

    hsmem = (hs0, hs1)
    rsmem = (rs0, rs1)
    tsmem = (ts0, ts1)
    hbufs = (hbuf0, hbuf1)
    tbufs = (tbuf0, tbuf1)
    rbufs = (rbuf0, rbuf1)
    nbufs = (nbuf0, nbuf1)
    sems = (sem0, sem1)

    nst = 512 // CHUNK
    nitems = 2 * nst
    row_bytes = D * 4

    def start(i):
        b = i % 2
        side = i // nst
        ci = i % nst
        hslot = 0 if side == 0 else 3
        blk = (ci * CHUNK) // 128
        off = (ci * CHUNK) % 128
        pltpu.sync_copy(shidx.at[sid, hslot, blk, pl.ds(off, CHUNK)], hsmem[b])
        pltpu.sync_copy(shidx.at[sid, hslot + 1, blk, pl.ds(off, CHUNK)], rsmem[b])
        pltpu.sync_copy(shidx.at[sid, hslot + 2, blk, pl.ds(off, CHUNK)], tsmem[b])

        def issue(j, _):
            hj = hsmem[b][j]
            rj = rsmem[b][j]
            tj = tsmem[b][j]
            pltpu.async_copy(ent_w.at[hj], hbufs[b].at[j], sems[b])
            pltpu.async_copy(ent_w.at[tj], tbufs[b].at[j], sems[b])
            pltpu.async_copy(rel_w.at[rj], rbufs[b].at[j], sems[b])
            pltpu.async_copy(norm_w.at[rj], nbufs[b].at[j], sems[b])
            return 0

        lax.fori_loop(0, CHUNK, issue, 0)

    def drain(i):
        b = i % 2

        def wait_one(j, _):
            pltpu.make_async_copy(ent_w.at[0], hbufs[b].at[0], sems[b]).wait()
            return 0

        lax.fori_loop(0, 4 * CHUNK, wait_one, 0)

    lanes = lax.iota(jnp.int32, LANES)
    zeros = jnp.zeros((LANES,), jnp.float32)
    start(0)
    for i in range(nitems):
        if i + 1 < nitems:
            start(i + 1)
        drain(i)
        b = i % 2
        hb, tb, rb, nb = hbufs[b], tbufs[b], rbufs[b], nbufs[b]
        side = i // nst
        ci = i % nst
        obuf = opos_v if side == 0 else oneg_v
        obase = ci * CHUNK

        def group(g, _, hb=hb, tb=tb, rb=rb, nb=nb, obuf=obuf, obase=obase):
            jrows = lanes + g * LANES

            def dblock(dblk, carry):
                a, c2, s = carry
                cbase = jnp.zeros((LANES,), jnp.int32) + dblk * UNROLL
                for k in range(UNROLL):
                    cols = cbase + k
                    hvv = plsc.load_gather(hb, [jrows, cols])
                    tvv = plsc.load_gather(tb, [jrows, cols])
                    rvv = plsc.load_gather(rb, [jrows, cols])
                    nvv = plsc.load_gather(nb, [jrows, cols])
                    u = hvv - tvv
                    w = u + rvv
                    a = a + w * w
                    c2 = c2 + w * nvv
                    s = s + u * nvv
                return (a, c2, s)

            a, c2, s = lax.fori_loop(0, D // UNROLL, dblock, (zeros,) * 3)
            res = a - s * (2.0 * c2 - s)
            obuf[obase // 128, pl.ds(obase % 128 + g * LANES, LANES)] = res
            return 0

        lax.fori_loop(0, groups, group, 0)

    pltpu.sync_copy(opos_v, pos_out.at[wid])
    pltpu.sync_copy(oneg_v, neg_out.at[wid])


def kernel(ph, pr, pt, nh, nr, nt, ent_w, rel_w, norm_w):
    B = ph.shape[0]
    rows_per_w = B // NW
    nblk = rows_per_w // 128

    mesh = plsc.VectorSubcoreMesh(core_axis_name="c", subcore_axis_name="s")
    run = functools.partial(
        pl.kernel,
        mesh=mesh,
        compiler_params=pltpu.CompilerParams(
            needs_layout_passes=False,
            use_tc_tiling_on_sc=True,
        ),
        out_type=[
            jax.ShapeDtypeStruct((NW, nblk, 128), jnp.float32),
            jax.ShapeDtypeStruct((NW, nblk, 128), jnp.float32),
        ],
        scratch_types=[
            pltpu.VMEM((nblk, 128), jnp.int32),
            pltpu.VMEM((nblk, 128), jnp.int32),
            pltpu.VMEM((nblk, 128), jnp.int32),
            pltpu.VMEM((nblk, 128), jnp.int32),
            pltpu.VMEM((nblk, 128), jnp.int32),
            pltpu.VMEM((nblk, 128), jnp.int32),
            pltpu.VMEM_SHARED((NS, 6, nblk, 128), jnp.int32),
            pltpu.SMEM((CHUNK,), jnp.int32),
            pltpu.SMEM((CHUNK,), jnp.int32),
            pltpu.SMEM((CHUNK,), jnp.int32),
            pltpu.SMEM((CHUNK,), jnp.int32),
            pltpu.SMEM((CHUNK,), jnp.int32),
            pltpu.SMEM((CHUNK,), jnp.int32),
            pltpu.VMEM((CHUNK, D), jnp.float32),
            pltpu.VMEM((CHUNK, D), jnp.float32),
            pltpu.VMEM((CHUNK, D), jnp.float32),
            pltpu.VMEM((CHUNK, D), jnp.float32),
            pltpu.VMEM((CHUNK, D), jnp.float32),
            pltpu.VMEM((CHUNK, D), jnp.float32),
            pltpu.VMEM((CHUNK, D), jnp.float32),
            pltpu.VMEM((CHUNK, D), jnp.float32),
            pltpu.VMEM((nblk, 128), jnp.float32),
            pltpu.VMEM((nblk, 128), jnp.float32),
            pltpu.SemaphoreType.DMA,
            pltpu.SemaphoreType.DMA,
        ],
    )(_tile_body)

    pos, neg = run(
        ph.reshape(NW, nblk, 128), pr.reshape(NW, nblk, 128),
        pt.reshape(NW, nblk, 128), nh.reshape(NW, nblk, 128),
        nr.reshape(NW, nblk, 128), nt.reshape(NW, nblk, 128),
        ent_w, rel_w, norm_w,
    )
    return pos.reshape(B), neg.reshape(B)
